# baseline (speedup 1.0000x reference)
import functools

import jax
import jax.numpy as jnp
from jax import lax
from jax.experimental import pallas as pl
from jax.experimental.pallas import tpu as pltpu
from jax.experimental.pallas import tpu_sc as plsc

_D0, _D1 = 345, 1024
_COLS = 512
_NP = _D1 // _COLS

_mesh = plsc.VectorSubcoreMesh(
    core_axis_name="c", subcore_axis_name="s", num_cores=1, num_subcores=16
)


@functools.partial(
    pl.kernel,
    mesh=_mesh,
    out_type=jax.ShapeDtypeStruct((_D0, _D1), jnp.float16),
    scratch_types=[pltpu.VMEM((_D0, _COLS), jnp.float16)],
)
def _copy_panels(para_hbm, out_hbm, buf):
    wid = lax.axis_index("s")

    @pl.when(wid < _NP)
    def _panel():
        base = pl.multiple_of(wid * _COLS, _COLS)
        pltpu.sync_copy(para_hbm.at[:, pl.ds(base, _COLS)], buf)
        pltpu.sync_copy(buf, out_hbm.at[:, pl.ds(base, _COLS)])


def kernel(para, classes):
    del classes
    return _copy_panels(para.reshape(_D0, _D1))


# speedup vs baseline: 1.2389x; 1.2389x over previous
import functools

import jax
import jax.numpy as jnp
from jax import lax
from jax.experimental import pallas as pl
from jax.experimental.pallas import tpu as pltpu
from jax.experimental.pallas import tpu_sc as plsc

_D0, _D1 = 345, 1024
_COLS = 128
_NP = _D1 // _COLS

_mesh = plsc.VectorSubcoreMesh(
    core_axis_name="c", subcore_axis_name="s", num_cores=1, num_subcores=8
)


@functools.partial(
    pl.kernel,
    mesh=_mesh,
    out_type=jax.ShapeDtypeStruct((_D0, _D1), jnp.float16),
    scratch_types=[pltpu.VMEM((_D0, _COLS), jnp.float16)],
)
def _copy_panels(para_hbm, out_hbm, buf):
    wid = lax.axis_index("s")

    @pl.when(wid < _NP)
    def _panel():
        base = pl.multiple_of(wid * _COLS, _COLS)
        pltpu.sync_copy(para_hbm.at[:, pl.ds(base, _COLS)], buf)
        pltpu.sync_copy(buf, out_hbm.at[:, pl.ds(base, _COLS)])


def kernel(para, classes):
    del classes
    return _copy_panels(para.reshape(_D0, _D1))


# speedup vs baseline: 1.2471x; 1.0067x over previous
import functools

import jax
import jax.numpy as jnp
from jax import lax
from jax.experimental import pallas as pl
from jax.experimental.pallas import tpu as pltpu
from jax.experimental.pallas import tpu_sc as plsc

_D0, _D1 = 345, 1024
_COLS = 128
_NP = _D1 // _COLS

_mesh = plsc.VectorSubcoreMesh(
    core_axis_name="c", subcore_axis_name="s", num_cores=1, num_subcores=_NP
)


@functools.partial(
    pl.kernel,
    mesh=_mesh,
    out_type=jax.ShapeDtypeStruct((_D0, _D1), jnp.float16),
    scratch_types=[pltpu.VMEM((_D0, _COLS), jnp.float16)],
)
def _copy_panels(para_hbm, out_hbm, buf):
    wid = lax.axis_index("s")
    base = pl.multiple_of(wid * _COLS, _COLS)
    pltpu.sync_copy(para_hbm.at[:, pl.ds(base, _COLS)], buf)
    pltpu.sync_copy(buf, out_hbm.at[:, pl.ds(base, _COLS)])


def kernel(para, classes):
    del classes
    return _copy_panels(para.reshape(_D0, _D1))
